# Initial kernel scaffold; baseline (speedup 1.0000x reference)
import jax
import jax.numpy as jnp
from jax.experimental import pallas as pl


def kernel(x, edge_index, W1, b1, W2, b2):
    raise NotImplementedError("write your pallas kernel here")


# speedup vs baseline: 3.6829x; 3.6829x over previous
import functools

import jax
import jax.numpy as jnp
from jax import lax
from jax.experimental import pallas as pl
from jax.experimental.pallas import tpu as pltpu
from jax.experimental.pallas import tpu_sc as plsc

N = 10000
E = 320000
D = 128

NC = 2
NS = 16
NW = NC * NS

CHUNK = 128
EDGES_PT = -(-E // (NW * CHUNK)) * CHUNK
E_PAD = EDGES_PT * NW
NCHUNK = EDGES_PT // CHUNK

ROWS_PT = -(-(N + 8) // (NS * 8)) * 8
AGG_ROWS = ROWS_PT * NS

_mesh = plsc.VectorSubcoreMesh(core_axis_name="c", subcore_axis_name="s")


@functools.partial(
    pl.kernel,
    out_type=jax.ShapeDtypeStruct((NC, AGG_ROWS, D), jnp.float32),
    mesh=_mesh,
    scratch_types=[
        pltpu.VMEM((CHUNK,), jnp.int32),
        pltpu.VMEM((CHUNK,), jnp.int32),
        pltpu.VMEM((CHUNK, D), jnp.float32),
        pltpu.VMEM_SHARED((AGG_ROWS, D), jnp.float32),
        pltpu.SemaphoreType.DMA,
    ],
)
def _sc_scatter(x_hbm, row_hbm, col_hbm, zeros_hbm, out_hbm,
                rowv, colv, gath, agg, sem):
    cid = lax.axis_index("c")
    sid = lax.axis_index("s")
    wid = cid * NS + sid

    pltpu.sync_copy(zeros_hbm, agg.at[pl.ds(sid * ROWS_PT, ROWS_PT)])
    plsc.subcore_barrier()

    base = wid * EDGES_PT

    def body(i, carry):
        off = base + i * CHUNK
        pltpu.sync_copy(row_hbm.at[pl.ds(off, CHUNK)], rowv)
        pltpu.sync_copy(col_hbm.at[pl.ds(off, CHUNK)], colv)
        pltpu.async_copy(x_hbm.at[colv], gath, sem).wait()
        pltpu.sync_copy(gath, agg.at[rowv], add=True)
        return carry

    lax.fori_loop(0, NCHUNK, body, 0)

    plsc.subcore_barrier()
    pltpu.sync_copy(agg.at[pl.ds(sid * ROWS_PT, ROWS_PT)],
                    out_hbm.at[cid, pl.ds(sid * ROWS_PT, ROWS_PT)])


_ROWS_BLK = 1000


def _mlp_body(x_ref, p_ref, w1_ref, b1_ref, w2_ref, b2_ref, o_ref):
    acc = x_ref[...] + p_ref[0] + p_ref[1]
    h = jnp.maximum(
        jnp.dot(acc, w1_ref[...], preferred_element_type=jnp.float32)
        + b1_ref[...], 0.0)
    o_ref[...] = (jnp.dot(h, w2_ref[...], preferred_element_type=jnp.float32)
                  + b2_ref[...])


_mlp = pl.pallas_call(
    _mlp_body,
    out_shape=jax.ShapeDtypeStruct((N, D), jnp.float32),
    grid=(N // _ROWS_BLK,),
    in_specs=[
        pl.BlockSpec((_ROWS_BLK, D), lambda i: (i, 0)),
        pl.BlockSpec((NC, _ROWS_BLK, D), lambda i: (0, i, 0)),
        pl.BlockSpec((D, D), lambda i: (0, 0)),
        pl.BlockSpec((1, D), lambda i: (0, 0)),
        pl.BlockSpec((D, D), lambda i: (0, 0)),
        pl.BlockSpec((1, D), lambda i: (0, 0)),
    ],
    out_specs=pl.BlockSpec((_ROWS_BLK, D), lambda i: (i, 0)),
)


def kernel(x, edge_index, W1, b1, W2, b2):
    row = edge_index[0].astype(jnp.int32)
    col = edge_index[1].astype(jnp.int32)
    npad = E_PAD - E
    row_p = jnp.concatenate([row, jnp.full((npad,), N, jnp.int32)])
    col_p = jnp.concatenate([col, jnp.zeros((npad,), jnp.int32)])
    zeros = jnp.zeros((ROWS_PT, D), jnp.float32)
    partials = _sc_scatter(x, row_p, col_p, zeros)
    return _mlp(x, partials, W1, b1.reshape(1, D), W2, b2.reshape(1, D))
